# Initial kernel scaffold; baseline (speedup 1.0000x reference)
import jax
import jax.numpy as jnp
from jax.experimental import pallas as pl


def kernel(seq_emb, pos_table):
    raise NotImplementedError("write your pallas kernel here")


# speedup vs baseline: 3.2056x; 3.2056x over previous
import jax
import jax.numpy as jnp
from jax.experimental import pallas as pl

_BLOCK_S = 256


def _add_kernel(seq_ref, pos_ref, out_ref):
    out_ref[...] = seq_ref[...] + pos_ref[...][None, :, :]


def kernel(seq_emb, pos_table):
    batch, seq_len, dim = seq_emb.shape
    grid = (seq_len // _BLOCK_S,)
    return pl.pallas_call(
        _add_kernel,
        grid=grid,
        in_specs=[
            pl.BlockSpec((batch, _BLOCK_S, dim), lambda i: (0, i, 0)),
            pl.BlockSpec((_BLOCK_S, dim), lambda i: (i, 0)),
        ],
        out_specs=pl.BlockSpec((batch, _BLOCK_S, dim), lambda i: (0, i, 0)),
        out_shape=jax.ShapeDtypeStruct((batch, seq_len, dim), seq_emb.dtype),
    )(seq_emb, pos_table)
